# baseline (speedup 1.0000x reference)
import jax
import jax.numpy as jnp
from jax.experimental import pallas as pl
from jax.experimental.pallas import tpu as pltpu

_BLOCK = 25000


def _mlp_kernel(f_ref, wlt_ref, blt_ref, w1_ref, b1_ref, w2_ref, b2_ref, o_ref):
    wc = jax.lax.dot_general(
        wlt_ref[...], w1_ref[...], (((0,), (1,)), ((), ())),
        preferred_element_type=jnp.float32)
    bc = jax.lax.dot_general(
        blt_ref[...], w1_ref[...], (((1,), (1,)), ((), ())),
        preferred_element_type=jnp.float32) + b1_ref[...]
    f = f_ref[...]
    h = jnp.dot(f, wc, preferred_element_type=jnp.float32) + bc
    h = jnp.maximum(h, 0.0)
    o_ref[...] = jax.lax.dot_general(
        h, w2_ref[...], (((1,), (1,)), ((), ())),
        preferred_element_type=jnp.float32) + b2_ref[...]


def kernel(features, points, nuv, ranges, W_lt, b_lt, W1, b1, W2, b2):
    del points, nuv, ranges
    n, d_in = features.shape
    d_out = W_lt.shape[0]
    weight_spec = lambda shape: pl.BlockSpec(shape, lambda i: (0, 0))
    return pl.pallas_call(
        _mlp_kernel,
        grid=(pl.cdiv(n, _BLOCK),),
        in_specs=[
            pl.BlockSpec((_BLOCK, d_in), lambda i: (i, 0)),
            weight_spec((d_out, d_in)),
            weight_spec((1, d_out)),
            weight_spec((d_out, d_out)),
            weight_spec((1, d_out)),
            weight_spec((d_out, d_out)),
            weight_spec((1, d_out)),
        ],
        out_specs=pl.BlockSpec((_BLOCK, d_out), lambda i: (i, 0)),
        out_shape=jax.ShapeDtypeStruct((n, d_out), jnp.float32),
        compiler_params=pltpu.CompilerParams(
            dimension_semantics=("parallel",),
        ),
    )(features, W_lt, b_lt[None, :], W1, b1[None, :], W2, b2[None, :])


# speedup vs baseline: 1.1278x; 1.1278x over previous
import jax
import jax.numpy as jnp
from jax.experimental import pallas as pl
from jax.experimental.pallas import tpu as pltpu

_BLOCK = 10000


def _mlp_kernel(f_ref, wlt_ref, blt_ref, w1_ref, b1_ref, w2_ref, b2_ref, o_ref):
    wc = jax.lax.dot_general(
        wlt_ref[...], w1_ref[...], (((0,), (1,)), ((), ())),
        preferred_element_type=jnp.float32)
    bc = jax.lax.dot_general(
        blt_ref[...], w1_ref[...], (((1,), (1,)), ((), ())),
        preferred_element_type=jnp.float32) + b1_ref[...]
    f = f_ref[...]
    h = jnp.dot(f, wc, preferred_element_type=jnp.float32) + bc
    h = jnp.maximum(h, 0.0)
    o_ref[...] = jax.lax.dot_general(
        h, w2_ref[...], (((1,), (1,)), ((), ())),
        preferred_element_type=jnp.float32) + b2_ref[...]


def kernel(features, points, nuv, ranges, W_lt, b_lt, W1, b1, W2, b2):
    del points, nuv, ranges
    n, d_in = features.shape
    d_out = W_lt.shape[0]
    weight_spec = lambda shape: pl.BlockSpec(shape, lambda i: (0, 0))
    return pl.pallas_call(
        _mlp_kernel,
        grid=(pl.cdiv(n, _BLOCK),),
        in_specs=[
            pl.BlockSpec((_BLOCK, d_in), lambda i: (i, 0)),
            weight_spec((d_out, d_in)),
            weight_spec((1, d_out)),
            weight_spec((d_out, d_out)),
            weight_spec((1, d_out)),
            weight_spec((d_out, d_out)),
            weight_spec((1, d_out)),
        ],
        out_specs=pl.BlockSpec((_BLOCK, d_out), lambda i: (i, 0)),
        out_shape=jax.ShapeDtypeStruct((n, d_out), jnp.float32),
        compiler_params=pltpu.CompilerParams(
            dimension_semantics=("parallel",),
        ),
    )(features, W_lt, b_lt[None, :], W1, b1[None, :], W2, b2[None, :])


# speedup vs baseline: 1.1438x; 1.0142x over previous
import jax
import jax.numpy as jnp
from jax.experimental import pallas as pl
from jax.experimental.pallas import tpu as pltpu

_BLOCK = 20000


def _mlp_kernel(f_ref, wlt_ref, blt_ref, w1_ref, b1_ref, w2_ref, b2_ref, o_ref):
    wc = jax.lax.dot_general(
        wlt_ref[...], w1_ref[...], (((0,), (1,)), ((), ())),
        preferred_element_type=jnp.float32)
    bc = jax.lax.dot_general(
        blt_ref[...], w1_ref[...], (((1,), (1,)), ((), ())),
        preferred_element_type=jnp.float32) + b1_ref[...]
    f = f_ref[...]
    h = jnp.dot(f, wc, preferred_element_type=jnp.float32) + bc
    h = jnp.maximum(h, 0.0)
    o_ref[...] = jax.lax.dot_general(
        h, w2_ref[...], (((1,), (1,)), ((), ())),
        preferred_element_type=jnp.float32) + b2_ref[...]


def kernel(features, points, nuv, ranges, W_lt, b_lt, W1, b1, W2, b2):
    del points, nuv, ranges
    n, d_in = features.shape
    d_out = W_lt.shape[0]
    weight_spec = lambda shape: pl.BlockSpec(shape, lambda i: (0, 0))
    return pl.pallas_call(
        _mlp_kernel,
        grid=(pl.cdiv(n, _BLOCK),),
        in_specs=[
            pl.BlockSpec((_BLOCK, d_in), lambda i: (i, 0)),
            weight_spec((d_out, d_in)),
            weight_spec((1, d_out)),
            weight_spec((d_out, d_out)),
            weight_spec((1, d_out)),
            weight_spec((d_out, d_out)),
            weight_spec((1, d_out)),
        ],
        out_specs=pl.BlockSpec((_BLOCK, d_out), lambda i: (i, 0)),
        out_shape=jax.ShapeDtypeStruct((n, d_out), jnp.float32),
        compiler_params=pltpu.CompilerParams(
            dimension_semantics=("parallel",),
        ),
    )(features, W_lt, b_lt[None, :], W1, b1[None, :], W2, b2[None, :])
